# baseline (speedup 1.0000x reference)
import jax
import jax.numpy as jnp
from jax import lax
from jax.experimental import pallas as pl

_RHO = 1e-08
_NPP = 8


def _fused_body(xs_ref, mem0_ref, mm1_ref, mm2_ref, out_ref):
    f32 = jnp.float32
    FB, B, D = xs_ref.shape
    M0 = mem0_ref.shape[2]
    H1 = mm1_ref.shape[2]
    H2 = mm2_ref.shape[2]
    npp = mm2_ref.shape[0]

    iota0 = lax.broadcasted_iota(jnp.int32, (M0, B), 0).astype(f32)
    iota1 = lax.broadcasted_iota(jnp.int32, (H1, B), 0).astype(f32)

    x = xs_ref[...]
    xn = jnp.sqrt(jnp.sum(x * x, axis=-1))

    def field_select(f):
        m = jnp.transpose(mem0_ref[f]) - 0.5
        num = lax.dot_general(m, x[f], (((1,), (1,)), ((), ())),
                              preferred_element_type=f32) * 0.5
        mn = jnp.sqrt(jnp.sum(m * m, axis=-1))
        h0 = num / (mn[:, None] * xn[f][None, :] + _RHO) + 0.5
        vmax = jnp.max(h0, axis=0)
        wm = jnp.where(h0 == vmax[None, :], iota0, float(M0))
        amin = jnp.min(wm, axis=0)
        sel = jnp.where(iota0 == amin[None, :], vmax[None, :], 0.0)
        return sel, vmax

    for j in range(npp):
        sel1 = []
        val1 = []
        for nl in range(4):
            prop = jnp.zeros((H1, B), f32)
            sumsq = jnp.zeros((1, B), f32)
            for c in range(4):
                sel, vmax = field_select(16 * j + 4 * nl + c)
                prop = prop + lax.dot_general(
                    mm1_ref[4 * j + nl, c], sel, (((1,), (0,)), ((), ())),
                    preferred_element_type=f32)
                sumsq = sumsq + (vmax * vmax)[None, :]
            coeff = 1.0 / (4.0 * jnp.sqrt(sumsq) + _RHO)
            h1 = prop * coeff
            v1 = jnp.max(h1, axis=0)
            wm1 = jnp.where(h1 == v1[None, :], iota1, float(H1))
            amin1 = jnp.min(wm1, axis=0)
            sel1.append(jnp.where(iota1 == amin1[None, :], v1[None, :], 0.0))
            val1.append(v1)

        prop2 = jnp.zeros((H2, B), f32)
        sumsq2 = jnp.zeros((1, B), f32)
        for c in range(4):
            prop2 = prop2 + lax.dot_general(
                mm2_ref[j, c], sel1[c], (((1,), (0,)), ((), ())),
                preferred_element_type=f32)
            sumsq2 = sumsq2 + (val1[c] * val1[c])[None, :]
        coeff2 = 1.0 / (4.0 * jnp.sqrt(sumsq2) + _RHO)
        out_ref[j] = prop2 * coeff2


@jax.jit
def kernel(xs, mem0, mm1, mm2):
    B, F, D = xs.shape
    M0 = mem0.shape[1]
    N1, C1, H1, _ = mm1.shape
    N2, C2, H2, _ = mm2.shape
    FB = (F // N2) * _NPP

    xs_r = jnp.transpose(xs, (1, 0, 2)) - 0.5
    mem0_t = jnp.transpose(mem0, (0, 2, 1))

    out = pl.pallas_call(
        _fused_body,
        grid=(N2 // _NPP,),
        in_specs=[
            pl.BlockSpec((FB, B, D), lambda i: (i, 0, 0)),
            pl.BlockSpec((FB, D, M0), lambda i: (i, 0, 0)),
            pl.BlockSpec((4 * _NPP, C1, H1, M0), lambda i: (i, 0, 0, 0)),
            pl.BlockSpec((_NPP, C2, H2, H1), lambda i: (i, 0, 0, 0)),
        ],
        out_specs=pl.BlockSpec((_NPP, H2, B), lambda i: (i, 0, 0)),
        out_shape=jax.ShapeDtypeStruct((N2, H2, B), jnp.float32),
    )(xs_r, mem0_t, mm1, mm2)
    return jnp.transpose(out, (2, 0, 1))


# speedup vs baseline: 1.0256x; 1.0256x over previous
import jax
import jax.numpy as jnp
from jax import lax
from jax.experimental import pallas as pl

_RHO = 1e-08
_NPP = 4


def _xs_transpose_body(xs_ref, out_ref):
    b, d, ft = xs_ref.shape
    flat = xs_ref[...].reshape(b * d, ft)
    out_ref[...] = jnp.transpose(flat) - 0.5


def _fused_body(xs_ref, mem0_ref, mm1_ref, mm2_ref, out_ref):
    f32 = jnp.float32
    FB, B, D = xs_ref.shape
    M0 = mem0_ref.shape[2]
    H1 = mm1_ref.shape[2]
    H2 = mm2_ref.shape[2]
    npp = mm2_ref.shape[0]

    iota0 = lax.broadcasted_iota(jnp.int32, (M0, B), 0).astype(f32)
    iota1 = lax.broadcasted_iota(jnp.int32, (H1, B), 0).astype(f32)

    x = xs_ref[...]
    xn = jnp.sqrt(jnp.sum(x * x, axis=-1))

    def field_select(f):
        m = jnp.transpose(mem0_ref[f]) - 0.5
        num = lax.dot_general(m, x[f], (((1,), (1,)), ((), ())),
                              preferred_element_type=f32) * 0.5
        mn = jnp.sqrt(jnp.sum(m * m, axis=-1))
        h0 = num / (mn[:, None] * xn[f][None, :] + _RHO) + 0.5
        vmax = jnp.max(h0, axis=0)
        wm = jnp.where(h0 == vmax[None, :], iota0, float(M0))
        amin = jnp.min(wm, axis=0)
        sel = jnp.where(iota0 == amin[None, :], vmax[None, :], 0.0)
        return sel, vmax

    for j in range(npp):
        sel1 = []
        val1 = []
        for nl in range(4):
            prop = jnp.zeros((H1, B), f32)
            sumsq = jnp.zeros((1, B), f32)
            for c in range(4):
                sel, vmax = field_select(16 * j + 4 * nl + c)
                prop = prop + lax.dot_general(
                    mm1_ref[4 * j + nl, c], sel, (((1,), (0,)), ((), ())),
                    preferred_element_type=f32)
                sumsq = sumsq + (vmax * vmax)[None, :]
            coeff = 1.0 / (4.0 * jnp.sqrt(sumsq) + _RHO)
            h1 = prop * coeff
            v1 = jnp.max(h1, axis=0)
            wm1 = jnp.where(h1 == v1[None, :], iota1, float(H1))
            amin1 = jnp.min(wm1, axis=0)
            sel1.append(jnp.where(iota1 == amin1[None, :], v1[None, :], 0.0))
            val1.append(v1)

        prop2 = jnp.zeros((H2, B), f32)
        sumsq2 = jnp.zeros((1, B), f32)
        for c in range(4):
            prop2 = prop2 + lax.dot_general(
                mm2_ref[j, c], sel1[c], (((1,), (0,)), ((), ())),
                preferred_element_type=f32)
            sumsq2 = sumsq2 + (val1[c] * val1[c])[None, :]
        coeff2 = 1.0 / (4.0 * jnp.sqrt(sumsq2) + _RHO)
        out_ref[j] = prop2 * coeff2


@jax.jit
def kernel(xs, mem0, mm1, mm2):
    B, F, D = xs.shape
    M0 = mem0.shape[1]
    N1, C1, H1, _ = mm1.shape
    N2, C2, H2, _ = mm2.shape
    FB = (F // N2) * _NPP

    FT = 128
    xs_b = jnp.transpose(xs, (0, 2, 1))
    xs_flat = pl.pallas_call(
        _xs_transpose_body,
        grid=(F // FT,),
        in_specs=[pl.BlockSpec((B, D, FT), lambda i: (0, 0, i))],
        out_specs=pl.BlockSpec((FT, B * D), lambda i: (i, 0)),
        out_shape=jax.ShapeDtypeStruct((F, B * D), jnp.float32),
    )(xs_b)
    xs_r = xs_flat.reshape(F, B, D)
    mem0_t = jnp.transpose(mem0, (0, 2, 1))

    out = pl.pallas_call(
        _fused_body,
        grid=(N2 // _NPP,),
        in_specs=[
            pl.BlockSpec((FB, B, D), lambda i: (i, 0, 0)),
            pl.BlockSpec((FB, D, M0), lambda i: (i, 0, 0)),
            pl.BlockSpec((4 * _NPP, C1, H1, M0), lambda i: (i, 0, 0, 0)),
            pl.BlockSpec((_NPP, C2, H2, H1), lambda i: (i, 0, 0, 0)),
        ],
        out_specs=pl.BlockSpec((_NPP, H2, B), lambda i: (i, 0, 0)),
        out_shape=jax.ShapeDtypeStruct((N2, H2, B), jnp.float32),
    )(xs_r, mem0_t, mm1, mm2)
    return jnp.transpose(out, (2, 0, 1))


# speedup vs baseline: 1.0268x; 1.0011x over previous
import jax
import jax.numpy as jnp
from jax import lax
from jax.experimental import pallas as pl

_RHO = 1e-08
_NPP = 8


def _xs_transpose_body(xs_ref, out_ref):
    b, d, ft = xs_ref.shape
    flat = xs_ref[...].reshape(b * d, ft)
    out_ref[...] = jnp.transpose(flat) - 0.5


def _fused_body(xs_ref, mem0_ref, mm1_ref, mm2_ref, out_ref):
    f32 = jnp.float32
    FB, B, D = xs_ref.shape
    M0 = mem0_ref.shape[2]
    H1 = mm1_ref.shape[2]
    H2 = mm2_ref.shape[2]
    npp = mm2_ref.shape[0]

    iota0 = lax.broadcasted_iota(jnp.int32, (M0, B), 0).astype(f32)
    iota1 = lax.broadcasted_iota(jnp.int32, (H1, B), 0).astype(f32)

    x = xs_ref[...]
    xn = jnp.sqrt(jnp.sum(x * x, axis=-1))

    def field_select(f):
        m = jnp.transpose(mem0_ref[f]) - 0.5
        num = lax.dot_general(m, x[f], (((1,), (1,)), ((), ())),
                              preferred_element_type=f32) * 0.5
        mn = jnp.sqrt(jnp.sum(m * m, axis=-1))
        h0 = num / (mn[:, None] * xn[f][None, :] + _RHO) + 0.5
        vmax = jnp.max(h0, axis=0)
        wm = jnp.where(h0 == vmax[None, :], iota0, float(M0))
        amin = jnp.min(wm, axis=0)
        sel = jnp.where(iota0 == amin[None, :], vmax[None, :], 0.0)
        return sel, vmax

    for j in range(npp):
        sel1 = []
        val1 = []
        for nl in range(4):
            prop = jnp.zeros((H1, B), f32)
            sumsq = jnp.zeros((1, B), f32)
            for c in range(4):
                sel, vmax = field_select(16 * j + 4 * nl + c)
                prop = prop + lax.dot_general(
                    mm1_ref[4 * j + nl, c], sel, (((1,), (0,)), ((), ())),
                    preferred_element_type=f32)
                sumsq = sumsq + (vmax * vmax)[None, :]
            coeff = 1.0 / (4.0 * jnp.sqrt(sumsq) + _RHO)
            h1 = prop * coeff
            v1 = jnp.max(h1, axis=0)
            wm1 = jnp.where(h1 == v1[None, :], iota1, float(H1))
            amin1 = jnp.min(wm1, axis=0)
            sel1.append(jnp.where(iota1 == amin1[None, :], v1[None, :], 0.0))
            val1.append(v1)

        prop2 = jnp.zeros((H2, B), f32)
        sumsq2 = jnp.zeros((1, B), f32)
        for c in range(4):
            prop2 = prop2 + lax.dot_general(
                mm2_ref[j, c], sel1[c], (((1,), (0,)), ((), ())),
                preferred_element_type=f32)
            sumsq2 = sumsq2 + (val1[c] * val1[c])[None, :]
        coeff2 = 1.0 / (4.0 * jnp.sqrt(sumsq2) + _RHO)
        out_ref[j] = prop2 * coeff2


@jax.jit
def kernel(xs, mem0, mm1, mm2):
    B, F, D = xs.shape
    M0 = mem0.shape[1]
    N1, C1, H1, _ = mm1.shape
    N2, C2, H2, _ = mm2.shape
    FB = (F // N2) * _NPP

    FT = 128
    xs_b = jnp.transpose(xs, (0, 2, 1))
    xs_flat = pl.pallas_call(
        _xs_transpose_body,
        grid=(F // FT,),
        in_specs=[pl.BlockSpec((B, D, FT), lambda i: (0, 0, i))],
        out_specs=pl.BlockSpec((FT, B * D), lambda i: (i, 0)),
        out_shape=jax.ShapeDtypeStruct((F, B * D), jnp.float32),
    )(xs_b)
    xs_r = xs_flat.reshape(F, B, D)
    mem0_t = jnp.transpose(mem0, (0, 2, 1))

    out = pl.pallas_call(
        _fused_body,
        grid=(N2 // _NPP,),
        in_specs=[
            pl.BlockSpec((FB, B, D), lambda i: (i, 0, 0)),
            pl.BlockSpec((FB, D, M0), lambda i: (i, 0, 0)),
            pl.BlockSpec((4 * _NPP, C1, H1, M0), lambda i: (i, 0, 0, 0)),
            pl.BlockSpec((_NPP, C2, H2, H1), lambda i: (i, 0, 0, 0)),
        ],
        out_specs=pl.BlockSpec((_NPP, H2, B), lambda i: (i, 0, 0)),
        out_shape=jax.ShapeDtypeStruct((N2, H2, B), jnp.float32),
    )(xs_r, mem0_t, mm1, mm2)
    return jnp.transpose(out, (2, 0, 1))
